# baseline (speedup 1.0000x reference)
import functools

import jax
import jax.numpy as jnp
from jax import lax
from jax.experimental import pallas as pl
from jax.experimental.pallas import tpu as pltpu
from jax.experimental.pallas import tpu_sc as plsc

B, S, D = 128, 8192, 64
V = 6
K = int(S * 0.15)
KPAD = 1232
L = 16
NCHUNK = S // L
NC, NS = 2, 16
NW = NC * NS
ROWS_PER_W = B // NW


def _sc_body(x_hbm, tbl_hbm, w1_hbm, w2_hbm, p1_hbm, p2_hbm, bias_hbm,
             idx_hbm, pred_hbm,
             xrow, chunkcnt, outidx, tbl, w1, p1, w2, p2, bias,
             score_s, totals_s, glob_r, nsel_s, pooled_r, pred_r,
             in_sem, out_sem):
    wid = lax.axis_index("s") * NC + lax.axis_index("c")
    iota = lax.iota(jnp.int32, L)

    pltpu.sync_copy(tbl_hbm, tbl)
    pltpu.sync_copy(w1_hbm, w1)
    pltpu.sync_copy(w2_hbm, w2)
    pltpu.sync_copy(p1_hbm, p1)
    pltpu.sync_copy(p2_hbm, p2)
    pltpu.sync_copy(bias_hbm, bias)

    def splat(ref, idx):
        return plsc.load_gather(ref, [jnp.full((L,), idx, jnp.int32)])

    def splat1(ref, idx):
        return plsc.load_gather(ref, [jnp.full((L,), idx + 1, jnp.int32)])

    def store1(ref, vec):
        plsc.store_scatter(ref, [iota + 1], vec)

    sv = jnp.zeros((L,), jnp.float32)
    for v in range(V):
        def mlp_step(d, acc):
            a0, a1 = acc
            t = splat(tbl, v * D + d)
            return (a0 + t * w1[d, pl.ds(0, L)], a1 + t * w1[d, pl.ds(L, L)])
        a0, a1 = lax.fori_loop(
            0, D, mlp_step, (bias[pl.ds(0, L)], bias[pl.ds(L, L)]))
        h0 = jnp.maximum(a0, 0.0)
        h1 = jnp.maximum(a1, 0.0)
        logit = jnp.sum(h0 * w2[pl.ds(0, L)] + h1 * w2[pl.ds(L, L)])
        sv = jnp.where(iota == v, logit, sv)
    store1(score_s, sv)

    @plsc.parallel_loop(0, NCHUNK, unroll=16)
    def _(c):
        chunkcnt[pl.ds(pl.multiple_of(c * L, L), L)] = jnp.zeros(
            (L,), jnp.int32)

    def hist_chunk(xbuf, c):
        xv = xbuf[pl.ds(pl.multiple_of(c * L, L), L)]
        cntv, lastm = plsc.scan_count(xv)
        plsc.addupdate_scatter(chunkcnt, [c * L + xv], cntv, mask=lastm)

    predvec = jnp.zeros((L,), jnp.float32)
    row0 = wid * ROWS_PER_W
    pin = [None] * (ROWS_PER_W + 1)
    pin[0] = pltpu.async_copy(x_hbm.at[row0], xrow.at[0], in_sem)
    pin[0].wait()
    pin[1] = pltpu.async_copy(x_hbm.at[row0 + 1], xrow.at[1], in_sem)
    pending_out = None

    @plsc.parallel_loop(0, NCHUNK, unroll=16)
    def _(c):
        hist_chunk(xrow.at[0], c)

    for i in range(ROWS_PER_W):
        row = row0 + i
        xbuf = xrow.at[i % 2]
        xnext = xrow.at[(i + 1) % 2]

        @plsc.parallel_loop(0, NCHUNK, unroll=8,
                            carry=jnp.zeros((L,), jnp.int32))
        def totals(c, run):
            off = pl.ds(pl.multiple_of(c * L, L), L)
            v = chunkcnt[off]
            chunkcnt[off] = run
            return run + v
        store1(totals_s, totals)

        glob = jnp.zeros((L,), jnp.int32)
        for u in range(V):
            su = splat1(score_s, u)
            tu = splat1(totals_s, u)
            before = (su > sv) | ((su == sv) & (u < iota))
            glob = glob + jnp.where(before, tu, 0)
        glob_r[...] = glob
        nsel = jnp.minimum(glob + totals, K) - jnp.minimum(glob, K)
        store1(nsel_s, nsel)

        if i + 1 < ROWS_PER_W:
            pin[i + 1].wait()
        if pending_out is not None:
            pending_out.wait()

        last_row = i + 1 == ROWS_PER_W

        @plsc.parallel_loop(0, NCHUNK, unroll=16)
        def _(c):
            off = pl.ds(pl.multiple_of(c * L, L), L)
            xv = xbuf[off]
            cntv, _lastm = plsc.scan_count(xv)
            cb = plsc.load_gather(chunkcnt, [c * L + xv])
            gb = plsc.load_gather(glob_r, [xv])
            chunkcnt[off] = jnp.zeros((L,), jnp.int32)
            dest = gb + cb + cntv - 1
            pos = iota + c * L
            plsc.store_scatter(outidx, [dest], pos, mask=dest < K)
            if not last_row:
                hist_chunk(xnext, c)

        pending_out = pltpu.async_copy(outidx, idx_hbm.at[row], out_sem)
        if i + 2 < ROWS_PER_W:
            pin[i + 2] = pltpu.async_copy(
                x_hbm.at[row + 2], xrow.at[i % 2], in_sem)

        p = [jnp.zeros((L,), jnp.float32) for _ in range(4)]
        for v in range(V):
            nf = splat1(nsel_s, v).astype(jnp.float32)
            p = [p[j] + nf * tbl[pl.ds(v * D + j * L, L)] for j in range(4)]
        kf = jnp.float32(K)
        for j in range(4):
            pooled_r[pl.ds(j * L, L)] = p[j] / kf

        def head_step(d, acc):
            s = splat(pooled_r, d)
            return tuple(acc[j] + s * p1[d, pl.ds(j * L, L)] for j in range(4))
        acc = lax.fori_loop(
            0, D, head_step,
            tuple(bias[pl.ds(32 + j * L, L)] for j in range(4)))
        ph = [jnp.maximum(a, 0.0) for a in acc]
        t = ph[0] * p2[pl.ds(0, L)]
        for j in range(1, 4):
            t = t + ph[j] * p2[pl.ds(j * L, L)]
        z = jnp.sum(t)
        zv = jnp.full((L,), z, jnp.float32) + splat(bias, 97)
        sig = 1.0 / (1.0 + jnp.exp(-zv))
        predvec = jnp.where(iota == i, sig, predvec)

    pending_out.wait()
    pred_r[...] = predvec
    pltpu.sync_copy(pred_r, pred_hbm.at[wid])


@jax.jit
def _run(x, table, W1, W2v, P1, P2v, bias):
    mesh = plsc.VectorSubcoreMesh(core_axis_name="c", subcore_axis_name="s")
    f = pl.kernel(
        _sc_body,
        out_type=(
            jax.ShapeDtypeStruct((B, KPAD), jnp.int32),
            jax.ShapeDtypeStruct((NW, L), jnp.float32),
        ),
        mesh=mesh,
        compiler_params=pltpu.CompilerParams(
            needs_layout_passes=False, use_tc_tiling_on_sc=False),
        scratch_types=[
            pltpu.VMEM((2, S), jnp.int32),
            pltpu.VMEM((NCHUNK * L,), jnp.int32),
            pltpu.VMEM((KPAD,), jnp.int32),
            pltpu.VMEM((V * D,), jnp.float32),
            pltpu.VMEM((D, 32), jnp.float32),
            pltpu.VMEM((D, D), jnp.float32),
            pltpu.VMEM((32,), jnp.float32),
            pltpu.VMEM((D,), jnp.float32),
            pltpu.VMEM((128,), jnp.float32),
            pltpu.VMEM((2 * L,), jnp.float32),
            pltpu.VMEM((2 * L,), jnp.int32),
            pltpu.VMEM((L,), jnp.int32),
            pltpu.VMEM((2 * L,), jnp.int32),
            pltpu.VMEM((D,), jnp.float32),
            pltpu.VMEM((L,), jnp.float32),
            pltpu.SemaphoreType.DMA,
            pltpu.SemaphoreType.DMA,
        ],
    )
    return f(x, table, W1, W2v, P1, P2v, bias)


def kernel(x, table, W1, b1, W2, b2, P1, pb1, P2, pb2):
    bias = jnp.concatenate(
        [b1, pb1, b2, pb2, jnp.zeros((30,), jnp.float32)])
    idx_pad, predbuf = _run(
        x.astype(jnp.int32), table.reshape(V * D), W1, W2.reshape(32), P1,
        P2.reshape(64), bias)
    prediction = predbuf[:, :ROWS_PER_W].reshape(B)
    indices = idx_pad[:, :K]
    return (prediction, indices)


# speedup vs baseline: 1.0503x; 1.0503x over previous
import functools

import jax
import jax.numpy as jnp
from jax import lax
from jax.experimental import pallas as pl
from jax.experimental.pallas import tpu as pltpu
from jax.experimental.pallas import tpu_sc as plsc

B, S, D = 128, 8192, 64
V = 6
K = int(S * 0.15)
KPAD = 1232
L = 16
NCHUNK = S // L
NC, NS = 2, 16
NW = NC * NS
ROWS_PER_W = B // NW


def _sc_body(x_hbm, tbl_hbm, w1_hbm, w2_hbm, p1_hbm, p2_hbm, bias_hbm,
             idx_hbm, pred_hbm,
             xrow, chunkcnt, outidx, tbl, w1, p1, w2, p2, bias,
             score_s, totals_s, glob_r, nsel_s, pooled_r, pred_r,
             in_sem, out_sem):
    wid = lax.axis_index("s") * NC + lax.axis_index("c")
    iota = lax.iota(jnp.int32, L)

    pltpu.sync_copy(tbl_hbm, tbl)
    pltpu.sync_copy(w1_hbm, w1)
    pltpu.sync_copy(w2_hbm, w2)
    pltpu.sync_copy(p1_hbm, p1)
    pltpu.sync_copy(p2_hbm, p2)
    pltpu.sync_copy(bias_hbm, bias)

    def splat(ref, idx):
        return plsc.load_gather(ref, [jnp.full((L,), idx, jnp.int32)])

    def splat1(ref, idx):
        return plsc.load_gather(ref, [jnp.full((L,), idx + 1, jnp.int32)])

    def store1(ref, vec):
        plsc.store_scatter(ref, [iota + 1], vec)

    sv = jnp.zeros((L,), jnp.float32)
    for v in range(V):
        def mlp_step(d, acc):
            a0, a1 = acc
            t = splat(tbl, v * D + d)
            return (a0 + t * w1[d, pl.ds(0, L)], a1 + t * w1[d, pl.ds(L, L)])
        a0, a1 = lax.fori_loop(
            0, D, mlp_step, (bias[pl.ds(0, L)], bias[pl.ds(L, L)]))
        h0 = jnp.maximum(a0, 0.0)
        h1 = jnp.maximum(a1, 0.0)
        logit = jnp.sum(h0 * w2[pl.ds(0, L)] + h1 * w2[pl.ds(L, L)])
        sv = jnp.where(iota == v, logit, sv)
    store1(score_s, sv)

    @plsc.parallel_loop(0, NCHUNK, unroll=16)
    def _(c):
        chunkcnt[pl.ds(pl.multiple_of(c * L, L), L)] = jnp.zeros(
            (L,), jnp.int32)

    def hist_chunk(xbuf, c):
        xv = xbuf[pl.ds(pl.multiple_of(c * L, L), L)]
        cntv, lastm = plsc.scan_count(xv)
        plsc.addupdate_scatter(chunkcnt, [c * L + xv], cntv, mask=lastm)

    predvec = jnp.zeros((L,), jnp.float32)
    row0 = wid * ROWS_PER_W
    pin = [None] * (ROWS_PER_W + 1)
    pin[0] = pltpu.async_copy(x_hbm.at[row0], xrow.at[0], in_sem)
    pin[0].wait()
    pin[1] = pltpu.async_copy(x_hbm.at[row0 + 1], xrow.at[1], in_sem)
    pending_out = None

    @plsc.parallel_loop(0, NCHUNK, unroll=16)
    def _(c):
        hist_chunk(xrow.at[0], c)

    for i in range(ROWS_PER_W):
        row = row0 + i
        xbuf = xrow.at[i % 2]
        xnext = xrow.at[(i + 1) % 2]

        @plsc.parallel_loop(0, NCHUNK, unroll=8,
                            carry=jnp.zeros((L,), jnp.int32))
        def totals(c, run):
            off = pl.ds(pl.multiple_of(c * L, L), L)
            v = chunkcnt[off]
            chunkcnt[off] = run
            return run + v
        store1(totals_s, totals)

        glob = jnp.zeros((L,), jnp.int32)
        for u in range(V):
            su = splat1(score_s, u)
            tu = splat1(totals_s, u)
            before = (su > sv) | ((su == sv) & (u < iota))
            glob = glob + jnp.where(before, tu, 0)
        glob_r[...] = glob
        nsel = jnp.minimum(glob + totals, K) - jnp.minimum(glob, K)
        store1(nsel_s, nsel)

        if i + 1 < ROWS_PER_W:
            pin[i + 1].wait()
        if pending_out is not None:
            pending_out.wait()

        last_row = i + 1 == ROWS_PER_W

        @plsc.parallel_loop(0, NCHUNK, unroll=8)
        def _(c):
            off = pl.ds(pl.multiple_of(c * L, L), L)
            xv = xbuf[off]
            cntv, _lastm = plsc.scan_count(xv)
            cb = plsc.load_gather(chunkcnt, [c * L + xv])
            gb = plsc.load_gather(glob_r, [xv])
            chunkcnt[off] = jnp.zeros((L,), jnp.int32)
            dest = gb + cb + cntv - 1
            pos = iota + c * L
            plsc.store_scatter(outidx, [dest], pos, mask=dest < K)
            if not last_row:
                hist_chunk(xnext, c)

        pending_out = pltpu.async_copy(outidx, idx_hbm.at[row], out_sem)
        if i + 2 < ROWS_PER_W:
            pin[i + 2] = pltpu.async_copy(
                x_hbm.at[row + 2], xrow.at[i % 2], in_sem)

        p = [jnp.zeros((L,), jnp.float32) for _ in range(4)]
        for v in range(V):
            nf = splat1(nsel_s, v).astype(jnp.float32)
            p = [p[j] + nf * tbl[pl.ds(v * D + j * L, L)] for j in range(4)]
        kf = jnp.float32(K)
        for j in range(4):
            pooled_r[pl.ds(j * L, L)] = p[j] / kf

        def head_step(d, acc):
            s = splat(pooled_r, d)
            return tuple(acc[j] + s * p1[d, pl.ds(j * L, L)] for j in range(4))
        acc = lax.fori_loop(
            0, D, head_step,
            tuple(bias[pl.ds(32 + j * L, L)] for j in range(4)))
        ph = [jnp.maximum(a, 0.0) for a in acc]
        t = ph[0] * p2[pl.ds(0, L)]
        for j in range(1, 4):
            t = t + ph[j] * p2[pl.ds(j * L, L)]
        z = jnp.sum(t)
        zv = jnp.full((L,), z, jnp.float32) + splat(bias, 97)
        sig = 1.0 / (1.0 + jnp.exp(-zv))
        predvec = jnp.where(iota == i, sig, predvec)

    pending_out.wait()
    pred_r[...] = predvec
    pltpu.sync_copy(pred_r, pred_hbm.at[wid])


@jax.jit
def _run(x, table, W1, W2v, P1, P2v, bias):
    mesh = plsc.VectorSubcoreMesh(core_axis_name="c", subcore_axis_name="s")
    f = pl.kernel(
        _sc_body,
        out_type=(
            jax.ShapeDtypeStruct((B, KPAD), jnp.int32),
            jax.ShapeDtypeStruct((NW, L), jnp.float32),
        ),
        mesh=mesh,
        compiler_params=pltpu.CompilerParams(
            needs_layout_passes=False, use_tc_tiling_on_sc=False),
        scratch_types=[
            pltpu.VMEM((2, S), jnp.int32),
            pltpu.VMEM((NCHUNK * L,), jnp.int32),
            pltpu.VMEM((KPAD,), jnp.int32),
            pltpu.VMEM((V * D,), jnp.float32),
            pltpu.VMEM((D, 32), jnp.float32),
            pltpu.VMEM((D, D), jnp.float32),
            pltpu.VMEM((32,), jnp.float32),
            pltpu.VMEM((D,), jnp.float32),
            pltpu.VMEM((128,), jnp.float32),
            pltpu.VMEM((2 * L,), jnp.float32),
            pltpu.VMEM((2 * L,), jnp.int32),
            pltpu.VMEM((L,), jnp.int32),
            pltpu.VMEM((2 * L,), jnp.int32),
            pltpu.VMEM((D,), jnp.float32),
            pltpu.VMEM((L,), jnp.float32),
            pltpu.SemaphoreType.DMA,
            pltpu.SemaphoreType.DMA,
        ],
    )
    return f(x, table, W1, W2v, P1, P2v, bias)


def kernel(x, table, W1, b1, W2, b2, P1, pb1, P2, pb2):
    bias = jnp.concatenate(
        [b1, pb1, b2, pb2, jnp.zeros((30,), jnp.float32)])
    idx_pad, predbuf = _run(
        x.astype(jnp.int32), table.reshape(V * D), W1, W2.reshape(32), P1,
        P2.reshape(64), bias)
    prediction = predbuf[:, :ROWS_PER_W].reshape(B)
    indices = idx_pad[:, :K]
    return (prediction, indices)


# speedup vs baseline: 1.0659x; 1.0148x over previous
import functools

import jax
import jax.numpy as jnp
from jax import lax
from jax.experimental import pallas as pl
from jax.experimental.pallas import tpu as pltpu
from jax.experimental.pallas import tpu_sc as plsc

B, S, D = 128, 8192, 64
V = 6
K = int(S * 0.15)
KPAD = 1232
L = 16
NCHUNK = S // L
NC, NS = 2, 16
NW = NC * NS
ROWS_PER_W = B // NW


def _sc_body(x_hbm, tbl_hbm, w1_hbm, w2_hbm, p1_hbm, p2_hbm, bias_hbm,
             idx_hbm, pred_hbm,
             xrow, chunkcnt, outidx, tbl, w1, p1, w2, p2, bias,
             score_s, totals_s, glob_r, nsel_s, pooled_r, pred2, pred_r,
             in_sem, out_sem):
    wid = lax.axis_index("s") * NC + lax.axis_index("c")
    iota = lax.iota(jnp.int32, L)

    pltpu.sync_copy(tbl_hbm, tbl)
    pltpu.sync_copy(w1_hbm, w1)
    pltpu.sync_copy(w2_hbm, w2)
    pltpu.sync_copy(p1_hbm, p1)
    pltpu.sync_copy(p2_hbm, p2)
    pltpu.sync_copy(bias_hbm, bias)

    def splat(ref, idx):
        return plsc.load_gather(ref, [jnp.full((L,), idx, jnp.int32)])

    def splat1(ref, idx):
        return plsc.load_gather(ref, [jnp.full((L,), idx + 1, jnp.int32)])

    def store1(ref, vec):
        plsc.store_scatter(ref, [iota + 1], vec)

    sv = jnp.zeros((L,), jnp.float32)
    for v in range(V):
        def mlp_step(d, acc):
            a0, a1 = acc
            t = splat(tbl, v * D + d)
            return (a0 + t * w1[d, pl.ds(0, L)], a1 + t * w1[d, pl.ds(L, L)])
        a0, a1 = lax.fori_loop(
            0, D, mlp_step, (bias[pl.ds(0, L)], bias[pl.ds(L, L)]))
        h0 = jnp.maximum(a0, 0.0)
        h1 = jnp.maximum(a1, 0.0)
        logit = jnp.sum(h0 * w2[pl.ds(0, L)] + h1 * w2[pl.ds(L, L)])
        sv = jnp.where(iota == v, logit, sv)
    store1(score_s, sv)

    def before_mask(u):
        su = splat1(score_s, u)
        return (su > sv) | ((su == sv) & (u < iota))

    predcnt = jnp.zeros((L,), jnp.int32)
    for u in range(V):
        predcnt = predcnt + jnp.where(before_mask(u), 1, 0)
    topm = (predcnt == 0) & (iota < V)
    top_s = jnp.sum(jnp.where(topm, iota, 0))

    def head_sig(pool):
        for j in range(4):
            pooled_r[pl.ds(j * L, L)] = pool[j]

        def head_step(d, acc):
            s = splat(pooled_r, d)
            return tuple(acc[j] + s * p1[d, pl.ds(j * L, L)] for j in range(4))
        acc = lax.fori_loop(
            0, D, head_step,
            tuple(bias[pl.ds(32 + j * L, L)] for j in range(4)))
        ph = [jnp.maximum(a, 0.0) for a in acc]
        t = ph[0] * p2[pl.ds(0, L)]
        for j in range(1, 4):
            t = t + ph[j] * p2[pl.ds(j * L, L)]
        z = jnp.sum(t)
        zv = jnp.full((L,), z, jnp.float32) + splat(bias, 97)
        return 1.0 / (1.0 + jnp.exp(-zv))

    sig_fast = head_sig([tbl[pl.ds(top_s * D + j * L, L)] for j in range(4)])

    @plsc.parallel_loop(0, NCHUNK, unroll=16)
    def _(c):
        chunkcnt[pl.ds(pl.multiple_of(c * L, L), L)] = jnp.zeros(
            (L,), jnp.int32)

    row0 = wid * ROWS_PER_W
    pending_in = pltpu.async_copy(x_hbm.at[row0], xrow.at[0], in_sem)
    pending_out = None
    for i in range(ROWS_PER_W):
        row = row0 + i
        xbuf = xrow.at[i % 2]
        pending_in.wait()
        if i + 1 < ROWS_PER_W:
            pending_in = pltpu.async_copy(
                x_hbm.at[row + 1], xrow.at[(i + 1) % 2], in_sem)
        if pending_out is not None:
            pending_out.wait()

        @plsc.parallel_loop(0, NCHUNK, unroll=8,
                            carry=jnp.zeros((L,), jnp.int32))
        def run_cnt(c, run):
            xv = xbuf[pl.ds(pl.multiple_of(c * L, L), L)]
            m = xv == top_s
            cum = plsc.cumsum(m.astype(jnp.int32))
            dest = run + cum - 1
            plsc.store_scatter(outidx, [jnp.maximum(dest, 0)], iota + c * L,
                               mask=m & (dest < K))
            return run + plsc.all_reduce_population_count(m)
        cnt_top = jnp.max(run_cnt)
        plsc.store_scatter(pred2, [jnp.full((L,), i + 1, jnp.int32)],
                           sig_fast, mask=iota == 0)

        @pl.when(cnt_top < K)
        def _():
            @plsc.parallel_loop(0, NCHUNK, unroll=16)
            def _(c):
                xv = xbuf[pl.ds(pl.multiple_of(c * L, L), L)]
                cntv, lastm = plsc.scan_count(xv)
                plsc.addupdate_scatter(
                    chunkcnt, [c * L + xv], cntv, mask=lastm)

            @plsc.parallel_loop(0, NCHUNK, unroll=8,
                                carry=jnp.zeros((L,), jnp.int32))
            def totals(c, run):
                off = pl.ds(pl.multiple_of(c * L, L), L)
                v = chunkcnt[off]
                chunkcnt[off] = run
                return run + v
            store1(totals_s, totals)

            glob = jnp.zeros((L,), jnp.int32)
            for u in range(V):
                glob = glob + jnp.where(before_mask(u), splat1(totals_s, u), 0)
            glob_r[...] = glob
            nsel = jnp.minimum(glob + totals, K) - jnp.minimum(glob, K)
            store1(nsel_s, nsel)

            @plsc.parallel_loop(0, NCHUNK, unroll=16)
            def _(c):
                off = pl.ds(pl.multiple_of(c * L, L), L)
                xv = xbuf[off]
                cntv, _lastm = plsc.scan_count(xv)
                cb = plsc.load_gather(chunkcnt, [c * L + xv])
                gb = plsc.load_gather(glob_r, [xv])
                chunkcnt[off] = jnp.zeros((L,), jnp.int32)
                dest = gb + cb + cntv - 1
                pos = iota + c * L
                plsc.store_scatter(outidx, [dest], pos, mask=dest < K)

            p = [jnp.zeros((L,), jnp.float32) for _ in range(4)]
            for v in range(V):
                nf = splat1(nsel_s, v).astype(jnp.float32)
                p = [p[j] + nf * tbl[pl.ds(v * D + j * L, L)]
                     for j in range(4)]
            kf = jnp.float32(K)
            sig = head_sig([p[j] / kf for j in range(4)])
            plsc.store_scatter(pred2, [jnp.full((L,), i + 1, jnp.int32)],
                               sig, mask=iota == 0)

        pending_out = pltpu.async_copy(outidx, idx_hbm.at[row], out_sem)

    pending_out.wait()
    pred_r[...] = plsc.load_gather(pred2, [iota + 1])
    pltpu.sync_copy(pred_r, pred_hbm.at[wid])


@jax.jit
def _run(x, table, W1, W2v, P1, P2v, bias):
    mesh = plsc.VectorSubcoreMesh(core_axis_name="c", subcore_axis_name="s")
    f = pl.kernel(
        _sc_body,
        out_type=(
            jax.ShapeDtypeStruct((B, KPAD), jnp.int32),
            jax.ShapeDtypeStruct((NW, L), jnp.float32),
        ),
        mesh=mesh,
        compiler_params=pltpu.CompilerParams(
            needs_layout_passes=False, use_tc_tiling_on_sc=False),
        scratch_types=[
            pltpu.VMEM((2, S), jnp.int32),
            pltpu.VMEM((NCHUNK * L,), jnp.int32),
            pltpu.VMEM((KPAD,), jnp.int32),
            pltpu.VMEM((V * D,), jnp.float32),
            pltpu.VMEM((D, 32), jnp.float32),
            pltpu.VMEM((D, D), jnp.float32),
            pltpu.VMEM((32,), jnp.float32),
            pltpu.VMEM((D,), jnp.float32),
            pltpu.VMEM((128,), jnp.float32),
            pltpu.VMEM((2 * L,), jnp.float32),
            pltpu.VMEM((2 * L,), jnp.int32),
            pltpu.VMEM((L,), jnp.int32),
            pltpu.VMEM((2 * L,), jnp.int32),
            pltpu.VMEM((D,), jnp.float32),
            pltpu.VMEM((2 * L,), jnp.float32),
            pltpu.VMEM((L,), jnp.float32),
            pltpu.SemaphoreType.DMA,
            pltpu.SemaphoreType.DMA,
        ],
    )
    return f(x, table, W1, W2v, P1, P2v, bias)


def kernel(x, table, W1, b1, W2, b2, P1, pb1, P2, pb2):
    bias = jnp.concatenate(
        [b1, pb1, b2, pb2, jnp.zeros((30,), jnp.float32)])
    idx_pad, predbuf = _run(
        x.astype(jnp.int32), table.reshape(V * D), W1, W2.reshape(32), P1,
        P2.reshape(64), bias)
    prediction = predbuf[:, :ROWS_PER_W].reshape(B)
    indices = idx_pad[:, :K]
    return (prediction, indices)


# speedup vs baseline: 1.0774x; 1.0108x over previous
import functools

import jax
import jax.numpy as jnp
from jax import lax
from jax.experimental import pallas as pl
from jax.experimental.pallas import tpu as pltpu
from jax.experimental.pallas import tpu_sc as plsc

B, S, D = 128, 8192, 64
V = 6
K = int(S * 0.15)
KPAD = 1232
L = 16
NCHUNK = S // L
NC, NS = 2, 16
NW = NC * NS
ROWS_PER_W = B // NW


def _sc_body(x_hbm, tbl_hbm, w1_hbm, w2_hbm, p1_hbm, p2_hbm, bias_hbm,
             idx_hbm, pred_hbm,
             xrow, chunkcnt, outidx, tbl, w1, p1, w2, p2, bias,
             score_s, totals_s, glob_r, nsel_s, pooled_r, pred2, pred_r,
             in_sem, out_sem):
    wid = lax.axis_index("s") * NC + lax.axis_index("c")
    iota = lax.iota(jnp.int32, L)

    pltpu.sync_copy(tbl_hbm, tbl)
    pltpu.sync_copy(w1_hbm, w1)
    pltpu.sync_copy(w2_hbm, w2)
    pltpu.sync_copy(p1_hbm, p1)
    pltpu.sync_copy(p2_hbm, p2)
    pltpu.sync_copy(bias_hbm, bias)

    def splat(ref, idx):
        return plsc.load_gather(ref, [jnp.full((L,), idx, jnp.int32)])

    def splat1(ref, idx):
        return plsc.load_gather(ref, [jnp.full((L,), idx + 1, jnp.int32)])

    def store1(ref, vec):
        plsc.store_scatter(ref, [iota + 1], vec)

    sv = jnp.zeros((L,), jnp.float32)
    for v in range(V):
        def mlp_step(d, acc):
            a0, a1 = acc
            t = splat(tbl, v * D + d)
            return (a0 + t * w1[d, pl.ds(0, L)], a1 + t * w1[d, pl.ds(L, L)])
        a0, a1 = lax.fori_loop(
            0, D, mlp_step, (bias[pl.ds(0, L)], bias[pl.ds(L, L)]))
        h0 = jnp.maximum(a0, 0.0)
        h1 = jnp.maximum(a1, 0.0)
        logit = jnp.sum(h0 * w2[pl.ds(0, L)] + h1 * w2[pl.ds(L, L)])
        sv = jnp.where(iota == v, logit, sv)
    store1(score_s, sv)

    def before_mask(u):
        su = splat1(score_s, u)
        return (su > sv) | ((su == sv) & (u < iota))

    predcnt = jnp.zeros((L,), jnp.int32)
    for u in range(V):
        predcnt = predcnt + jnp.where(before_mask(u), 1, 0)
    topm = (predcnt == 0) & (iota < V)
    top_s = jnp.sum(jnp.where(topm, iota, 0))

    def head_sig(pool):
        for j in range(4):
            pooled_r[pl.ds(j * L, L)] = pool[j]

        def head_step(d, acc):
            s = splat(pooled_r, d)
            return tuple(acc[j] + s * p1[d, pl.ds(j * L, L)] for j in range(4))
        acc = lax.fori_loop(
            0, D, head_step,
            tuple(bias[pl.ds(32 + j * L, L)] for j in range(4)))
        ph = [jnp.maximum(a, 0.0) for a in acc]
        t = ph[0] * p2[pl.ds(0, L)]
        for j in range(1, 4):
            t = t + ph[j] * p2[pl.ds(j * L, L)]
        z = jnp.sum(t)
        zv = jnp.full((L,), z, jnp.float32) + splat(bias, 97)
        return 1.0 / (1.0 + jnp.exp(-zv))

    sig_fast = head_sig([tbl[pl.ds(top_s * D + j * L, L)] for j in range(4)])

    @plsc.parallel_loop(0, NCHUNK, unroll=16)
    def _(c):
        chunkcnt[pl.ds(pl.multiple_of(c * L, L), L)] = jnp.zeros(
            (L,), jnp.int32)

    row0 = wid * ROWS_PER_W
    pending_in = pltpu.async_copy(x_hbm.at[row0], xrow.at[0], in_sem)
    pending_out = None
    for i in range(ROWS_PER_W):
        row = row0 + i
        xbuf = xrow.at[i % 2]
        pending_in.wait()
        if i + 1 < ROWS_PER_W:
            pending_in = pltpu.async_copy(
                x_hbm.at[row + 1], xrow.at[(i + 1) % 2], in_sem)
        if i % 2 == 0 and pending_out is not None:
            pending_out.wait()

        obase = (i % 2) * K

        @plsc.parallel_loop(0, NCHUNK, unroll=8,
                            carry=jnp.zeros((L,), jnp.int32))
        def run_cnt(c, run):
            xv = xbuf[pl.ds(pl.multiple_of(c * L, L), L)]
            m = xv == top_s
            cum = plsc.cumsum(m.astype(jnp.int32))
            dest = run + cum - 1
            plsc.store_scatter(outidx, [jnp.maximum(dest, 0) + obase],
                               iota + c * L, mask=m & (dest < K))
            return run + plsc.all_reduce_population_count(m)
        cnt_top = jnp.max(run_cnt)
        plsc.store_scatter(pred2, [jnp.full((L,), i + 1, jnp.int32)],
                           sig_fast, mask=iota == 0)

        @pl.when(cnt_top < K)
        def _():
            @plsc.parallel_loop(0, NCHUNK, unroll=16)
            def _(c):
                xv = xbuf[pl.ds(pl.multiple_of(c * L, L), L)]
                cntv, lastm = plsc.scan_count(xv)
                plsc.addupdate_scatter(
                    chunkcnt, [c * L + xv], cntv, mask=lastm)

            @plsc.parallel_loop(0, NCHUNK, unroll=8,
                                carry=jnp.zeros((L,), jnp.int32))
            def totals(c, run):
                off = pl.ds(pl.multiple_of(c * L, L), L)
                v = chunkcnt[off]
                chunkcnt[off] = run
                return run + v
            store1(totals_s, totals)

            glob = jnp.zeros((L,), jnp.int32)
            for u in range(V):
                glob = glob + jnp.where(before_mask(u), splat1(totals_s, u), 0)
            glob_r[...] = glob
            nsel = jnp.minimum(glob + totals, K) - jnp.minimum(glob, K)
            store1(nsel_s, nsel)

            @plsc.parallel_loop(0, NCHUNK, unroll=16)
            def _(c):
                off = pl.ds(pl.multiple_of(c * L, L), L)
                xv = xbuf[off]
                cntv, _lastm = plsc.scan_count(xv)
                cb = plsc.load_gather(chunkcnt, [c * L + xv])
                gb = plsc.load_gather(glob_r, [xv])
                chunkcnt[off] = jnp.zeros((L,), jnp.int32)
                dest = gb + cb + cntv - 1
                pos = iota + c * L
                plsc.store_scatter(outidx, [jnp.minimum(dest, K - 1) + obase],
                                   pos, mask=dest < K)

            p = [jnp.zeros((L,), jnp.float32) for _ in range(4)]
            for v in range(V):
                nf = splat1(nsel_s, v).astype(jnp.float32)
                p = [p[j] + nf * tbl[pl.ds(v * D + j * L, L)]
                     for j in range(4)]
            kf = jnp.float32(K)
            sig = head_sig([p[j] / kf for j in range(4)])
            plsc.store_scatter(pred2, [jnp.full((L,), i + 1, jnp.int32)],
                               sig, mask=iota == 0)

        if i % 2 == 1:
            pending_out = pltpu.async_copy(
                outidx, idx_hbm.at[pl.ds((row - 1) * K, 2 * K)], out_sem)

    pending_out.wait()
    pred_r[...] = plsc.load_gather(pred2, [iota + 1])
    pltpu.sync_copy(pred_r, pred_hbm.at[wid])


@jax.jit
def _run(x, table, W1, W2v, P1, P2v, bias):
    mesh = plsc.VectorSubcoreMesh(core_axis_name="c", subcore_axis_name="s")
    f = pl.kernel(
        _sc_body,
        out_type=(
            jax.ShapeDtypeStruct((B * K,), jnp.int32),
            jax.ShapeDtypeStruct((NW, L), jnp.float32),
        ),
        mesh=mesh,
        compiler_params=pltpu.CompilerParams(
            needs_layout_passes=False, use_tc_tiling_on_sc=False),
        scratch_types=[
            pltpu.VMEM((2, S), jnp.int32),
            pltpu.VMEM((NCHUNK * L,), jnp.int32),
            pltpu.VMEM((2 * K,), jnp.int32),
            pltpu.VMEM((V * D,), jnp.float32),
            pltpu.VMEM((D, 32), jnp.float32),
            pltpu.VMEM((D, D), jnp.float32),
            pltpu.VMEM((32,), jnp.float32),
            pltpu.VMEM((D,), jnp.float32),
            pltpu.VMEM((128,), jnp.float32),
            pltpu.VMEM((2 * L,), jnp.float32),
            pltpu.VMEM((2 * L,), jnp.int32),
            pltpu.VMEM((L,), jnp.int32),
            pltpu.VMEM((2 * L,), jnp.int32),
            pltpu.VMEM((D,), jnp.float32),
            pltpu.VMEM((2 * L,), jnp.float32),
            pltpu.VMEM((L,), jnp.float32),
            pltpu.SemaphoreType.DMA,
            pltpu.SemaphoreType.DMA,
        ],
    )
    return f(x, table, W1, W2v, P1, P2v, bias)


def kernel(x, table, W1, b1, W2, b2, P1, pb1, P2, pb2):
    bias = jnp.concatenate(
        [b1, pb1, b2, pb2, jnp.zeros((30,), jnp.float32)])
    idx_pad, predbuf = _run(
        x.astype(jnp.int32), table.reshape(V * D), W1, W2.reshape(32), P1,
        P2.reshape(64), bias)
    prediction = predbuf[:, :ROWS_PER_W].reshape(B)
    indices = idx_pad.reshape(B, K)
    return (prediction, indices)


# speedup vs baseline: 1.1200x; 1.0395x over previous
import functools

import jax
import jax.numpy as jnp
from jax import lax
from jax.experimental import pallas as pl
from jax.experimental.pallas import tpu as pltpu
from jax.experimental.pallas import tpu_sc as plsc

B, S, D = 128, 8192, 64
V = 6
K = int(S * 0.15)
KPAD = 1232
L = 16
NCHUNK = S // L
NC, NS = 2, 16
NW = NC * NS
ROWS_PER_W = B // NW


def _sc_body(x_hbm, tbl_hbm, w1_hbm, w2_hbm, p1_hbm, p2_hbm, bias_hbm,
             idx_hbm, pred_hbm,
             xrow, chunkcnt, outidx, tbl, w1, p1, w2, p2, bias,
             score_s, totals_s, glob_r, nsel_s, pooled_r, pred2, pred_r,
             in_sem, out_sem):
    wid = lax.axis_index("s") * NC + lax.axis_index("c")
    iota = lax.iota(jnp.int32, L)

    wcopies = [
        pltpu.async_copy(src, dst, out_sem)
        for src, dst in ((tbl_hbm, tbl), (w1_hbm, w1), (w2_hbm, w2),
                         (p1_hbm, p1), (p2_hbm, p2), (bias_hbm, bias))]
    for cp in wcopies:
        cp.wait()

    def splat(ref, idx):
        return plsc.load_gather(ref, [jnp.full((L,), idx, jnp.int32)])

    def splat1(ref, idx):
        return plsc.load_gather(ref, [jnp.full((L,), idx + 1, jnp.int32)])

    def store1(ref, vec):
        plsc.store_scatter(ref, [iota + 1], vec)

    sv = jnp.zeros((L,), jnp.float32)
    for v in range(V):
        def mlp_step(d, acc):
            a0, a1 = acc
            t = splat(tbl, v * D + d)
            return (a0 + t * w1[d, pl.ds(0, L)], a1 + t * w1[d, pl.ds(L, L)])
        a0, a1 = lax.fori_loop(
            0, D, mlp_step, (bias[pl.ds(0, L)], bias[pl.ds(L, L)]))
        h0 = jnp.maximum(a0, 0.0)
        h1 = jnp.maximum(a1, 0.0)
        logit = jnp.sum(h0 * w2[pl.ds(0, L)] + h1 * w2[pl.ds(L, L)])
        sv = jnp.where(iota == v, logit, sv)
    store1(score_s, sv)

    def before_mask(u):
        su = splat1(score_s, u)
        return (su > sv) | ((su == sv) & (u < iota))

    predcnt = jnp.zeros((L,), jnp.int32)
    for u in range(V):
        predcnt = predcnt + jnp.where(before_mask(u), 1, 0)
    topm = (predcnt == 0) & (iota < V)
    top_s = jnp.sum(jnp.where(topm, iota, 0))

    def head_sig(pool):
        for j in range(4):
            pooled_r[pl.ds(j * L, L)] = pool[j]

        def head_step(d, acc):
            s = splat(pooled_r, d)
            return tuple(acc[j] + s * p1[d, pl.ds(j * L, L)] for j in range(4))
        acc = lax.fori_loop(
            0, D, head_step,
            tuple(bias[pl.ds(32 + j * L, L)] for j in range(4)))
        ph = [jnp.maximum(a, 0.0) for a in acc]
        t = ph[0] * p2[pl.ds(0, L)]
        for j in range(1, 4):
            t = t + ph[j] * p2[pl.ds(j * L, L)]
        z = jnp.sum(t)
        zv = jnp.full((L,), z, jnp.float32) + splat(bias, 97)
        return 1.0 / (1.0 + jnp.exp(-zv))

    sig_fast = head_sig([tbl[pl.ds(top_s * D + j * L, L)] for j in range(4)])

    @plsc.parallel_loop(0, NCHUNK, unroll=16)
    def _(c):
        chunkcnt[pl.ds(pl.multiple_of(c * L, L), L)] = jnp.zeros(
            (L,), jnp.int32)

    row0 = wid * ROWS_PER_W
    pending_in = pltpu.async_copy(x_hbm.at[row0], xrow.at[0], in_sem)
    pending_out = None
    for i in range(ROWS_PER_W):
        row = row0 + i
        xbuf = xrow.at[i % 2]
        pending_in.wait()
        if i + 1 < ROWS_PER_W:
            pending_in = pltpu.async_copy(
                x_hbm.at[row + 1], xrow.at[(i + 1) % 2], in_sem)
        if i % 2 == 0 and pending_out is not None:
            pending_out.wait()

        obase = (i % 2) * K

        @plsc.parallel_loop(0, NCHUNK, unroll=8,
                            carry=jnp.zeros((L,), jnp.int32))
        def run_cnt(c, run):
            xv = xbuf[pl.ds(pl.multiple_of(c * L, L), L)]
            m = xv == top_s
            cum = plsc.cumsum(m.astype(jnp.int32))
            dest = run + cum - 1
            plsc.store_scatter(outidx, [jnp.maximum(dest, 0) + obase],
                               iota + c * L, mask=m & (dest < K))
            return run + plsc.all_reduce_population_count(m)
        cnt_top = jnp.max(run_cnt)
        plsc.store_scatter(pred2, [jnp.full((L,), i + 1, jnp.int32)],
                           sig_fast, mask=iota == 0)

        @pl.when(cnt_top < K)
        def _():
            @plsc.parallel_loop(0, NCHUNK, unroll=16)
            def _(c):
                xv = xbuf[pl.ds(pl.multiple_of(c * L, L), L)]
                cntv, lastm = plsc.scan_count(xv)
                plsc.addupdate_scatter(
                    chunkcnt, [c * L + xv], cntv, mask=lastm)

            @plsc.parallel_loop(0, NCHUNK, unroll=8,
                                carry=jnp.zeros((L,), jnp.int32))
            def totals(c, run):
                off = pl.ds(pl.multiple_of(c * L, L), L)
                v = chunkcnt[off]
                chunkcnt[off] = run
                return run + v
            store1(totals_s, totals)

            glob = jnp.zeros((L,), jnp.int32)
            for u in range(V):
                glob = glob + jnp.where(before_mask(u), splat1(totals_s, u), 0)
            glob_r[...] = glob
            nsel = jnp.minimum(glob + totals, K) - jnp.minimum(glob, K)
            store1(nsel_s, nsel)

            @plsc.parallel_loop(0, NCHUNK, unroll=16)
            def _(c):
                off = pl.ds(pl.multiple_of(c * L, L), L)
                xv = xbuf[off]
                cntv, _lastm = plsc.scan_count(xv)
                cb = plsc.load_gather(chunkcnt, [c * L + xv])
                gb = plsc.load_gather(glob_r, [xv])
                chunkcnt[off] = jnp.zeros((L,), jnp.int32)
                dest = gb + cb + cntv - 1
                pos = iota + c * L
                plsc.store_scatter(outidx, [jnp.minimum(dest, K - 1) + obase],
                                   pos, mask=dest < K)

            p = [jnp.zeros((L,), jnp.float32) for _ in range(4)]
            for v in range(V):
                nf = splat1(nsel_s, v).astype(jnp.float32)
                p = [p[j] + nf * tbl[pl.ds(v * D + j * L, L)]
                     for j in range(4)]
            kf = jnp.float32(K)
            sig = head_sig([p[j] / kf for j in range(4)])
            plsc.store_scatter(pred2, [jnp.full((L,), i + 1, jnp.int32)],
                               sig, mask=iota == 0)

        if i % 2 == 1:
            pending_out = pltpu.async_copy(
                outidx, idx_hbm.at[pl.ds((row - 1) * K, 2 * K)], out_sem)

    pending_out.wait()
    pred_r[...] = plsc.load_gather(pred2, [iota + 1])
    pltpu.sync_copy(pred_r, pred_hbm.at[wid])


@jax.jit
def _run(x, table, W1, W2v, P1, P2v, bias):
    mesh = plsc.VectorSubcoreMesh(core_axis_name="c", subcore_axis_name="s")
    f = pl.kernel(
        _sc_body,
        out_type=(
            jax.ShapeDtypeStruct((B * K,), jnp.int32),
            jax.ShapeDtypeStruct((NW, L), jnp.float32),
        ),
        mesh=mesh,
        compiler_params=pltpu.CompilerParams(
            needs_layout_passes=False, use_tc_tiling_on_sc=False),
        scratch_types=[
            pltpu.VMEM((2, S), jnp.int32),
            pltpu.VMEM((NCHUNK * L,), jnp.int32),
            pltpu.VMEM((2 * K,), jnp.int32),
            pltpu.VMEM((V * D,), jnp.float32),
            pltpu.VMEM((D, 32), jnp.float32),
            pltpu.VMEM((D, D), jnp.float32),
            pltpu.VMEM((32,), jnp.float32),
            pltpu.VMEM((D,), jnp.float32),
            pltpu.VMEM((128,), jnp.float32),
            pltpu.VMEM((2 * L,), jnp.float32),
            pltpu.VMEM((2 * L,), jnp.int32),
            pltpu.VMEM((L,), jnp.int32),
            pltpu.VMEM((2 * L,), jnp.int32),
            pltpu.VMEM((D,), jnp.float32),
            pltpu.VMEM((2 * L,), jnp.float32),
            pltpu.VMEM((L,), jnp.float32),
            pltpu.SemaphoreType.DMA,
            pltpu.SemaphoreType.DMA,
        ],
    )
    return f(x, table, W1, W2v, P1, P2v, bias)


def kernel(x, table, W1, b1, W2, b2, P1, pb1, P2, pb2):
    bias = jnp.concatenate(
        [b1, pb1, b2, pb2, jnp.zeros((30,), jnp.float32)])
    idx_pad, predbuf = _run(
        x.astype(jnp.int32), table.reshape(V * D), W1, W2.reshape(32), P1,
        P2.reshape(64), bias)
    prediction = predbuf[:, :ROWS_PER_W].reshape(B)
    indices = idx_pad.reshape(B, K)
    return (prediction, indices)


# speedup vs baseline: 1.1367x; 1.0149x over previous
import functools

import jax
import jax.numpy as jnp
from jax import lax
from jax.experimental import pallas as pl
from jax.experimental.pallas import tpu as pltpu
from jax.experimental.pallas import tpu_sc as plsc

B, S, D = 128, 8192, 64
V = 6
K = int(S * 0.15)
KPAD = 1232
L = 16
NCHUNK = S // L
NC, NS = 2, 16
NW = NC * NS
ROWS_PER_W = B // NW


def _sc_body(x_hbm, tbl_hbm, w1_hbm, w2_hbm, p1_hbm, p2_hbm, bias_hbm,
             idx_hbm, pred_hbm,
             xrow, chunkcnt, outidx, tbl, w1, p1, w2, p2, bias,
             score_s, totals_s, glob_r, nsel_s, pooled_r, pred2, pred_r,
             in_sem, out_sem):
    wid = lax.axis_index("s") * NC + lax.axis_index("c")
    iota = lax.iota(jnp.int32, L)

    row0 = wid * ROWS_PER_W
    pending_in = pltpu.async_copy(x_hbm.at[row0], xrow.at[0], in_sem)
    wcopies = [
        pltpu.async_copy(src, dst, out_sem)
        for src, dst in ((tbl_hbm, tbl), (w1_hbm, w1), (w2_hbm, w2),
                         (p1_hbm, p1), (p2_hbm, p2), (bias_hbm, bias))]
    for cp in wcopies:
        cp.wait()

    def splat(ref, idx):
        return plsc.load_gather(ref, [jnp.full((L,), idx, jnp.int32)])

    def splat1(ref, idx):
        return plsc.load_gather(ref, [jnp.full((L,), idx + 1, jnp.int32)])

    def store1(ref, vec):
        plsc.store_scatter(ref, [iota + 1], vec)

    sv = jnp.zeros((L,), jnp.float32)
    for v in range(V):
        def mlp_step(d, acc):
            a0, a1 = acc
            t = splat(tbl, v * D + d)
            return (a0 + t * w1[d, pl.ds(0, L)], a1 + t * w1[d, pl.ds(L, L)])
        a0, a1 = lax.fori_loop(
            0, D, mlp_step, (bias[pl.ds(0, L)], bias[pl.ds(L, L)]))
        h0 = jnp.maximum(a0, 0.0)
        h1 = jnp.maximum(a1, 0.0)
        logit = jnp.sum(h0 * w2[pl.ds(0, L)] + h1 * w2[pl.ds(L, L)])
        sv = jnp.where(iota == v, logit, sv)
    store1(score_s, sv)

    def before_mask(u):
        su = splat1(score_s, u)
        return (su > sv) | ((su == sv) & (u < iota))

    predcnt = jnp.zeros((L,), jnp.int32)
    for u in range(V):
        predcnt = predcnt + jnp.where(before_mask(u), 1, 0)
    topm = (predcnt == 0) & (iota < V)
    top_s = jnp.sum(jnp.where(topm, iota, 0))

    def head_sig(pool):
        for j in range(4):
            pooled_r[pl.ds(j * L, L)] = pool[j]

        def head_step(d, acc):
            s = splat(pooled_r, d)
            return tuple(acc[j] + s * p1[d, pl.ds(j * L, L)] for j in range(4))
        acc = lax.fori_loop(
            0, D, head_step,
            tuple(bias[pl.ds(32 + j * L, L)] for j in range(4)))
        ph = [jnp.maximum(a, 0.0) for a in acc]
        t = ph[0] * p2[pl.ds(0, L)]
        for j in range(1, 4):
            t = t + ph[j] * p2[pl.ds(j * L, L)]
        z = jnp.sum(t)
        zv = jnp.full((L,), z, jnp.float32) + splat(bias, 97)
        return 1.0 / (1.0 + jnp.exp(-zv))

    sig_fast = head_sig([tbl[pl.ds(top_s * D + j * L, L)] for j in range(4)])

    @plsc.parallel_loop(0, NCHUNK, unroll=16)
    def _(c):
        chunkcnt[pl.ds(pl.multiple_of(c * L, L), L)] = jnp.zeros(
            (L,), jnp.int32)

    pending_out = None
    for i in range(ROWS_PER_W):
        row = row0 + i
        xbuf = xrow.at[i % 2]
        pending_in.wait()
        if i + 1 < ROWS_PER_W:
            pending_in = pltpu.async_copy(
                x_hbm.at[row + 1], xrow.at[(i + 1) % 2], in_sem)
        if i % 2 == 0 and pending_out is not None:
            pending_out.wait()

        obase = (i % 2) * K

        @plsc.parallel_loop(0, NCHUNK, unroll=8,
                            carry=jnp.zeros((L,), jnp.int32))
        def run_cnt(c, run):
            xv = xbuf[pl.ds(pl.multiple_of(c * L, L), L)]
            m = xv == top_s
            cum = plsc.cumsum(m.astype(jnp.int32))
            dest = run + cum - 1
            plsc.store_scatter(outidx, [jnp.maximum(dest, 0) + obase],
                               iota + c * L, mask=m & (dest < K))
            return run + plsc.all_reduce_population_count(m)
        cnt_top = jnp.max(run_cnt)
        plsc.store_scatter(pred2, [jnp.full((L,), i + 1, jnp.int32)],
                           sig_fast, mask=iota == 0)

        @pl.when(cnt_top < K)
        def _():
            @plsc.parallel_loop(0, NCHUNK, unroll=16)
            def _(c):
                xv = xbuf[pl.ds(pl.multiple_of(c * L, L), L)]
                cntv, lastm = plsc.scan_count(xv)
                plsc.addupdate_scatter(
                    chunkcnt, [c * L + xv], cntv, mask=lastm)

            @plsc.parallel_loop(0, NCHUNK, unroll=8,
                                carry=jnp.zeros((L,), jnp.int32))
            def totals(c, run):
                off = pl.ds(pl.multiple_of(c * L, L), L)
                v = chunkcnt[off]
                chunkcnt[off] = run
                return run + v
            store1(totals_s, totals)

            glob = jnp.zeros((L,), jnp.int32)
            for u in range(V):
                glob = glob + jnp.where(before_mask(u), splat1(totals_s, u), 0)
            glob_r[...] = glob
            nsel = jnp.minimum(glob + totals, K) - jnp.minimum(glob, K)
            store1(nsel_s, nsel)

            @plsc.parallel_loop(0, NCHUNK, unroll=16)
            def _(c):
                off = pl.ds(pl.multiple_of(c * L, L), L)
                xv = xbuf[off]
                cntv, _lastm = plsc.scan_count(xv)
                cb = plsc.load_gather(chunkcnt, [c * L + xv])
                gb = plsc.load_gather(glob_r, [xv])
                chunkcnt[off] = jnp.zeros((L,), jnp.int32)
                dest = gb + cb + cntv - 1
                pos = iota + c * L
                plsc.store_scatter(outidx, [jnp.minimum(dest, K - 1) + obase],
                                   pos, mask=dest < K)

            p = [jnp.zeros((L,), jnp.float32) for _ in range(4)]
            for v in range(V):
                nf = splat1(nsel_s, v).astype(jnp.float32)
                p = [p[j] + nf * tbl[pl.ds(v * D + j * L, L)]
                     for j in range(4)]
            kf = jnp.float32(K)
            sig = head_sig([p[j] / kf for j in range(4)])
            plsc.store_scatter(pred2, [jnp.full((L,), i + 1, jnp.int32)],
                               sig, mask=iota == 0)

        if i % 2 == 1:
            pending_out = pltpu.async_copy(
                outidx, idx_hbm.at[pl.ds((row - 1) * K, 2 * K)], out_sem)

    pending_out.wait()
    pred_r[...] = plsc.load_gather(pred2, [iota + 1])
    pltpu.sync_copy(pred_r, pred_hbm.at[wid])


@jax.jit
def _run(x, table, W1, W2v, P1, P2v, bias):
    mesh = plsc.VectorSubcoreMesh(core_axis_name="c", subcore_axis_name="s")
    f = pl.kernel(
        _sc_body,
        out_type=(
            jax.ShapeDtypeStruct((B * K,), jnp.int32),
            jax.ShapeDtypeStruct((NW, L), jnp.float32),
        ),
        mesh=mesh,
        compiler_params=pltpu.CompilerParams(
            needs_layout_passes=False, use_tc_tiling_on_sc=False),
        scratch_types=[
            pltpu.VMEM((2, S), jnp.int32),
            pltpu.VMEM((NCHUNK * L,), jnp.int32),
            pltpu.VMEM((2 * K,), jnp.int32),
            pltpu.VMEM((V * D,), jnp.float32),
            pltpu.VMEM((D, 32), jnp.float32),
            pltpu.VMEM((D, D), jnp.float32),
            pltpu.VMEM((32,), jnp.float32),
            pltpu.VMEM((D,), jnp.float32),
            pltpu.VMEM((128,), jnp.float32),
            pltpu.VMEM((2 * L,), jnp.float32),
            pltpu.VMEM((2 * L,), jnp.int32),
            pltpu.VMEM((L,), jnp.int32),
            pltpu.VMEM((2 * L,), jnp.int32),
            pltpu.VMEM((D,), jnp.float32),
            pltpu.VMEM((2 * L,), jnp.float32),
            pltpu.VMEM((L,), jnp.float32),
            pltpu.SemaphoreType.DMA,
            pltpu.SemaphoreType.DMA,
        ],
    )
    return f(x, table, W1, W2v, P1, P2v, bias)


def kernel(x, table, W1, b1, W2, b2, P1, pb1, P2, pb2):
    bias = jnp.concatenate(
        [b1, pb1, b2, pb2, jnp.zeros((30,), jnp.float32)])
    idx_pad, predbuf = _run(
        x.astype(jnp.int32), table.reshape(V * D), W1, W2.reshape(32), P1,
        P2.reshape(64), bias)
    prediction = predbuf[:, :ROWS_PER_W].reshape(B)
    indices = idx_pad.reshape(B, K)
    return (prediction, indices)
